# baseline (speedup 1.0000x reference)
import functools

import jax
import jax.numpy as jnp
from jax import lax
from jax.experimental import pallas as pl
from jax.experimental.pallas import tpu as pltpu
from jax.experimental.pallas import tpu_sc as plsc

D = 256
K = 512
BLK = 1024
PAD = -1
CW = 0.25


def _tc_body(flat_ref, mask_ref, embed_ref, idxg_ref, idxo_ref, loss_ref):
    i = pl.program_id(0)
    f = flat_ref[...]
    e = embed_ref[...]
    fg = lax.dot_general(f, e, (((1,), (1,)), ((), ())),
                         preferred_element_type=jnp.float32)
    f2 = jnp.sum(f * f, axis=1, keepdims=True)
    e2 = jnp.sum(e * e, axis=1)
    dist = f2 - 2.0 * fg + e2[None, :]
    md = jnp.min(dist, axis=1, keepdims=True)
    iota = lax.broadcasted_iota(jnp.int32, (BLK, K), 1)
    idx = jnp.min(jnp.where(dist <= md, iota, K), axis=1)
    idx2 = idx.reshape(BLK // 128, 128)
    m = mask_ref[...] > 0
    idxg_ref[...] = jnp.where(m, idx2, K)
    idxo_ref[...] = jnp.where(m, idx2, PAD)
    s = jnp.sum(md)

    @pl.when(i == 0)
    def _():
        loss_ref[...] = jnp.zeros_like(loss_ref)

    loss_ref[...] += s


def _tc_stage(flat, mask2d, embed):
    rows = flat.shape[0]
    nblk = rows // BLK
    sub = BLK // 128
    return pl.pallas_call(
        _tc_body,
        grid=(nblk,),
        in_specs=[
            pl.BlockSpec((BLK, D), lambda i: (i, 0)),
            pl.BlockSpec((sub, 128), lambda i: (i, 0)),
            pl.BlockSpec((K, D), lambda i: (0, 0)),
        ],
        out_specs=[
            pl.BlockSpec((sub, 128), lambda i: (i, 0)),
            pl.BlockSpec((sub, 128), lambda i: (i, 0)),
            pl.BlockSpec((1, 1), lambda i: (0, 0)),
        ],
        out_shape=[
            jax.ShapeDtypeStruct((rows // 128, 128), jnp.int32),
            jax.ShapeDtypeStruct((rows // 128, 128), jnp.int32),
            jax.ShapeDtypeStruct((1, 1), jnp.float32),
        ],
    )(flat, mask2d, embed)


def _sc_gather(table, idx2d, rows):
    info = plsc.get_sparse_core_info()
    nw = info.num_cores * info.num_subcores
    per_w = rows // nw
    chunks = per_w // 128
    mesh = plsc.VectorSubcoreMesh(core_axis_name="c", subcore_axis_name="s")

    @functools.partial(
        pl.kernel,
        mesh=mesh,
        out_type=jax.ShapeDtypeStruct((rows, D), jnp.float32),
        scratch_types=[
            pltpu.VMEM((chunks, 128), jnp.int32),
            pltpu.VMEM((128, D), jnp.float32),
            pltpu.VMEM((128, D), jnp.float32),
            pltpu.SemaphoreType.DMA,
            pltpu.SemaphoreType.DMA,
        ],
    )
    def k(table_hbm, idx_hbm, out_hbm, idx_v, rows0, rows1, sem0, sem1):
        wid = lax.axis_index("s") * info.num_cores + lax.axis_index("c")
        pltpu.sync_copy(idx_hbm.at[pl.ds(wid * chunks, chunks)], idx_v)
        bufs = (rows0, rows1)
        sems = (sem0, sem1)
        pending = pltpu.async_copy(table_hbm.at[idx_v.at[0]], bufs[0], sems[0])
        for j in range(chunks):
            nxt = None
            if j + 1 < chunks:
                nxt = pltpu.async_copy(
                    table_hbm.at[idx_v.at[j + 1]], bufs[(j + 1) % 2],
                    sems[(j + 1) % 2])
            pending.wait()
            base = wid * per_w + j * 128
            pltpu.sync_copy(bufs[j % 2], out_hbm.at[pl.ds(base, 128)])
            pending = nxt

    return k(table, idx2d)


def kernel(z, phoneme_mask, embed):
    B, N, Dz = z.shape
    rows = B * N
    flat = z.reshape(rows, Dz)
    mask2d = phoneme_mask.reshape(rows // 128, 128).astype(jnp.int32)
    idxg, idxo, loss = _tc_stage(flat, mask2d, embed)
    table = jnp.concatenate([embed, jnp.zeros((1, Dz), jnp.float32)], axis=0)
    quant = _sc_gather(table, idxg, rows)
    quantize = quant.reshape(B, N, Dz)
    indices = idxo.reshape(B, N)
    commit_loss = loss[0, 0] * (CW / (rows * Dz))
    return (quantize, indices, commit_loss)
